# baseline (speedup 1.0000x reference)
import functools

import jax
import jax.numpy as jnp
from jax import lax
from jax.experimental import pallas as pl
from jax.experimental.pallas import tpu as pltpu
from jax.experimental.pallas import tpu_sc as plsc

_D = 9
_TBL = 144 * _D
_BK = 2


def _sc_lookup(h2, m2, table_flat, b, l):
    mesh = plsc.VectorSubcoreMesh(core_axis_name="c", subcore_axis_name="s")

    @functools.partial(
        pl.kernel,
        out_type=jax.ShapeDtypeStruct((b, l, _D), jnp.float32),
        mesh=mesh,
        scratch_types=[
            pltpu.VMEM((_TBL,), jnp.float32),
        ],
        compiler_params=pltpu.CompilerParams(
            use_tc_tiling_on_sc=True, needs_layout_passes=False
        ),
    )
    def k(h_hbm, m_hbm, t_hbm, o_hbm, t_v):
        pltpu.sync_copy(t_hbm, t_v)

        iota = lax.iota(jnp.int32, 16)
        zeros = iota * 0
        jv = [zeros + j for j in range(_D)]

        def body(h_vmem, m_vmem, o_vmem):
            for bb in range(_BK):
                bv = zeros + bb

                def chunk(s, lvec, bv=bv, bb=bb):
                    sl = pl.ds(s, 16)
                    c9 = h_vmem[bb, sl] * (6 * _D) + m_vmem[bb, sl] * _D
                    for j in range(_D):
                        vals = plsc.load_gather(t_v, [c9 + j])
                        plsc.store_scatter(o_vmem, [bv, lvec, jv[j]], vals)

                @pl.loop(0, (l // 16))
                def _(u):
                    chunk(u * 16, iota + u * 16)

                if l % 16:
                    chunk(l - 16, iota + (l - 16))

        pltpu.emit_pipeline(
            body,
            grid=(b // _BK,),
            in_specs=[
                pl.BlockSpec((_BK, l), lambda i: (i, 0)),
                pl.BlockSpec((_BK, l), lambda i: (i, 0)),
            ],
            out_specs=[pl.BlockSpec((_BK, l, _D), lambda i: (i, 0, 0))],
            core_axis_name=("c", "s"),
            dimension_semantics=(pltpu.PARALLEL,),
        )(h_hbm, m_hbm, o_hbm)

    return k(h2, m2, table_flat)


def kernel(hour_idx, minute_idx, minute_table, hour_table):
    b, l = hour_idx.shape
    h2 = hour_idx.astype(jnp.int32)
    m2 = minute_idx.astype(jnp.int32)
    n_minute = minute_table.shape[0]
    n_hour = hour_table.shape[0]
    table = jnp.concatenate(
        [
            jnp.repeat(hour_table, n_minute, axis=0),
            jnp.tile(minute_table, (n_hour, 1)),
        ],
        axis=1,
    ).reshape(-1)
    return _sc_lookup(h2, m2, table, b, l)


# speedup vs baseline: 1.2501x; 1.2501x over previous
import functools

import jax
import jax.numpy as jnp
from jax import lax
from jax.experimental import pallas as pl
from jax.experimental.pallas import tpu as pltpu
from jax.experimental.pallas import tpu_sc as plsc

_D = 9
_TBL = 144 * _D


def _sc_lookup(h2, m2, table_flat, b, l):
    mesh = plsc.VectorSubcoreMesh(core_axis_name="c", subcore_axis_name="s")

    @functools.partial(
        pl.kernel,
        out_type=jax.ShapeDtypeStruct((b, l, _D), jnp.float32),
        mesh=mesh,
        scratch_types=[
            pltpu.VMEM((_TBL,), jnp.float32),
        ],
        compiler_params=pltpu.CompilerParams(
            use_tc_tiling_on_sc=True, needs_layout_passes=False
        ),
    )
    def k(h_hbm, m_hbm, t_hbm, o_hbm, t_v):
        pltpu.sync_copy(t_hbm, t_v)

        iota = lax.iota(jnp.int32, 16)
        zeros = iota * 0
        jv = [zeros + j for j in range(_D)]

        def body(h_vmem, m_vmem, o_vmem):
            def chunk(s, lvec):
                sl = pl.ds(s, 16)
                c9 = h_vmem[0, sl] * (6 * _D) + m_vmem[0, sl] * _D
                for j in range(_D):
                    vals = plsc.load_gather(t_v, [c9 + j])
                    plsc.store_scatter(o_vmem, [zeros, lvec, jv[j]], vals)

            @pl.loop(0, (l // 16))
            def _(u):
                chunk(u * 16, iota + u * 16)

            if l % 16:
                chunk(l - 16, iota + (l - 16))

        pltpu.emit_pipeline(
            body,
            grid=(b,),
            in_specs=[
                pl.BlockSpec((1, l), lambda i: (i, 0)),
                pl.BlockSpec((1, l), lambda i: (i, 0)),
            ],
            out_specs=[pl.BlockSpec((1, l, _D), lambda i: (i, 0, 0))],
            core_axis_name=("c", "s"),
            dimension_semantics=(pltpu.PARALLEL,),
        )(h_hbm, m_hbm, o_hbm)

    return k(h2, m2, table_flat)


def kernel(hour_idx, minute_idx, minute_table, hour_table):
    b, l = hour_idx.shape
    h2 = hour_idx.astype(jnp.int32)
    m2 = minute_idx.astype(jnp.int32)
    n_minute = minute_table.shape[0]
    n_hour = hour_table.shape[0]
    table = jnp.concatenate(
        [
            jnp.repeat(hour_table, n_minute, axis=0),
            jnp.tile(minute_table, (n_hour, 1)),
        ],
        axis=1,
    ).reshape(-1)
    return _sc_lookup(h2, m2, table, b, l)
